# baseline (speedup 1.0000x reference)
import functools

import jax
import jax.numpy as jnp
from jax import lax
from jax.experimental import pallas as pl
from jax.experimental.pallas import tpu as pltpu
from jax.experimental.pallas import tpu_sc as plsc

DIM = 64
LANES = 16
CHUNK = 128


def kernel(centers, contexts, in_embed, out_embed):
    B = centers.shape[0]
    NC, NS = 2, 16
    NW = NC * NS
    b_per_w = B // NW
    n_chunks = b_per_w // CHUNK
    n_blocks = b_per_w // LANES

    centers2 = centers.reshape(B // CHUNK, CHUNK).astype(jnp.int32)
    contexts2 = contexts.reshape(B // CHUNK, CHUNK).astype(jnp.int32)

    mesh = plsc.VectorSubcoreMesh(core_axis_name="c", subcore_axis_name="s")

    @functools.partial(
        pl.kernel,
        out_type=jax.ShapeDtypeStruct((B,), jnp.float32),
        mesh=mesh,
        compiler_params=pltpu.CompilerParams(
            needs_layout_passes=False, use_tc_tiling_on_sc=False
        ),
        scratch_types=[
            pltpu.VMEM((n_chunks, CHUNK), jnp.int32),
            pltpu.VMEM((n_chunks, CHUNK), jnp.int32),
            pltpu.VMEM((b_per_w, DIM), jnp.float32),
            pltpu.VMEM((b_per_w, DIM), jnp.float32),
            pltpu.VMEM((b_per_w,), jnp.float32),
            pltpu.SemaphoreType.DMA,
            pltpu.SemaphoreType.DMA,
        ],
    )
    def _k(c_hbm, x_hbm, vtab_hbm, utab_hbm, o_hbm,
           cidx, xidx, vrows, urows, obuf, sem_v, sem_u):
        wid = lax.axis_index("s") * NC + lax.axis_index("c")
        base_chunk = wid * n_chunks

        pltpu.sync_copy(c_hbm.at[pl.ds(base_chunk, n_chunks)], cidx)
        pltpu.sync_copy(x_hbm.at[pl.ds(base_chunk, n_chunks)], xidx)

        copies = []
        for j in range(n_chunks):
            dst = pl.ds(j * CHUNK, CHUNK)
            copies.append(pltpu.async_copy(vtab_hbm.at[cidx.at[j]], vrows.at[dst], sem_v))
            copies.append(pltpu.async_copy(utab_hbm.at[xidx.at[j]], urows.at[dst], sem_u))
        for cp in copies:
            cp.wait()

        iota = lax.iota(jnp.int32, LANES)

        def block(g, carry):
            row0 = pl.multiple_of(g * LANES, LANES)
            tot = jnp.zeros((LANES,), jnp.float32)
            for r in range(LANES):
                row = row0 + r
                acc = vrows[row, pl.ds(0, LANES)] * urows[row, pl.ds(0, LANES)]
                for c in range(1, DIM // LANES):
                    acc = acc + vrows[row, pl.ds(c * LANES, LANES)] * urows[row, pl.ds(c * LANES, LANES)]
                tot = jnp.where(iota == r, jnp.sum(acc), tot)
            obuf[pl.ds(row0, LANES)] = tot
            return carry

        lax.fori_loop(0, n_blocks, block, 0)

        pltpu.sync_copy(obuf, o_hbm.at[pl.ds(wid * b_per_w, b_per_w)])

    return _k(centers2, contexts2, in_embed, out_embed)
